# baseline (device time: 318446 ns/iter reference)
import jax
import jax.numpy as jnp
from jax import lax
from jax.experimental import pallas as pl
from jax.experimental.pallas import tpu as pltpu


_N_CHUNK = 16


def _exchange(logits):
    t, v = logits.shape
    half = t // 2
    rows = half // _N_CHUNK

    def body(l_ref, o_ref, x_send, x_recv, y_send, y_recv):
        my_x = lax.axis_index("x")
        my_y = lax.axis_index("y")
        xn = (1 - my_x, my_y)
        yn = (my_x, 1 - my_y)

        barrier = pltpu.get_barrier_semaphore()
        for nbr in (xn, yn):
            pl.semaphore_signal(
                barrier, inc=1, device_id=nbr, device_id_type=pl.DeviceIdType.MESH
            )
        pl.semaphore_wait(barrier, 2)

        base = my_y * half

        x_rdmas = []
        for c in range(_N_CHUNK):
            r = pl.ds(base + c * rows, rows)
            rd = pltpu.make_async_remote_copy(
                src_ref=l_ref.at[r],
                dst_ref=o_ref.at[r],
                send_sem=x_send.at[c],
                recv_sem=x_recv.at[c],
                device_id=xn,
                device_id_type=pl.DeviceIdType.MESH,
            )
            rd.start()
            x_rdmas.append(rd)

        y_rdmas = []
        for c in range(_N_CHUNK):
            x_rdmas[c].wait_recv()
            r = pl.ds(base + c * rows, rows)
            rd = pltpu.make_async_remote_copy(
                src_ref=o_ref.at[r],
                dst_ref=o_ref.at[r],
                send_sem=y_send.at[c],
                recv_sem=y_recv.at[c],
                device_id=yn,
                device_id_type=pl.DeviceIdType.MESH,
            )
            rd.start()
            y_rdmas.append(rd)

        for c in range(_N_CHUNK):
            y_rdmas[c].wait_recv()
            x_rdmas[c].wait_send()
            y_rdmas[c].wait_send()

    return pl.pallas_call(
        body,
        out_shape=jax.ShapeDtypeStruct((t, v), jnp.float32),
        in_specs=[pl.BlockSpec(memory_space=pl.ANY)],
        out_specs=pl.BlockSpec(memory_space=pl.ANY),
        scratch_shapes=[
            pltpu.SemaphoreType.DMA((_N_CHUNK,)),
            pltpu.SemaphoreType.DMA((_N_CHUNK,)),
            pltpu.SemaphoreType.DMA((_N_CHUNK,)),
            pltpu.SemaphoreType.DMA((_N_CHUNK,)),
        ],
        compiler_params=pltpu.CompilerParams(collective_id=0),
    )(logits)


def _gemm_exchange(x, W):
    t, d = x.shape
    v = W.shape[1]
    half = t // 2
    cw = v // _N_CHUNK
    J = _N_CHUNK

    def body(x_ref, w_ref, l_out, o_out, acc, cp_sem, x_send, x_recv, y_send, y_recv):
        j = pl.program_id(0)
        my_x = lax.axis_index("x")
        my_y = lax.axis_index("y")
        xn = (1 - my_x, my_y)
        yn = (my_x, 1 - my_y)
        base = my_y * half
        rs = pl.ds(base, half)

        @pl.when(j == 0)
        def _():
            barrier = pltpu.get_barrier_semaphore()
            for nbr in (xn, yn):
                pl.semaphore_signal(
                    barrier, inc=1, device_id=nbr,
                    device_id_type=pl.DeviceIdType.MESH,
                )
            pl.semaphore_wait(barrier, 2)

        acc[...] = jnp.dot(
            x_ref[...], w_ref[...].astype(jnp.bfloat16),
            preferred_element_type=jnp.float32,
        ).astype(jnp.bfloat16)
        cp = pltpu.make_async_copy(acc, l_out.at[j], cp_sem)
        cp.start()
        cp.wait()

        def x_rdma(k):
            return pltpu.make_async_remote_copy(
                src_ref=l_out.at[k, rs],
                dst_ref=o_out.at[k, rs],
                send_sem=x_send.at[k],
                recv_sem=x_recv.at[k],
                device_id=xn,
                device_id_type=pl.DeviceIdType.MESH,
            )

        def y_rdma(k):
            return pltpu.make_async_remote_copy(
                src_ref=o_out.at[k, rs],
                dst_ref=o_out.at[k, rs],
                send_sem=y_send.at[k],
                recv_sem=y_recv.at[k],
                device_id=yn,
                device_id_type=pl.DeviceIdType.MESH,
            )

        x_rdma(j).start()

        @pl.when(j >= 2)
        def _():
            k = j - 2
            x_rdma(k).wait_recv()
            y_rdma(k).start()

        @pl.when(j == J - 1)
        def _():
            for k in (J - 2, J - 1):
                x_rdma(k).wait_recv()
                y_rdma(k).start()
            for k in range(J):
                y_rdma(k).wait_recv()
                x_rdma(k).wait_send()
                y_rdma(k).wait_send()

    return pl.pallas_call(
        body,
        grid=(J,),
        out_shape=(
            jax.ShapeDtypeStruct((J, t, cw), jnp.bfloat16),
            jax.ShapeDtypeStruct((J, t, cw), jnp.bfloat16),
        ),
        in_specs=[
            pl.BlockSpec((t, d), lambda j: (0, 0)),
            pl.BlockSpec((d, cw), lambda j: (0, j)),
        ],
        out_specs=(
            pl.BlockSpec(memory_space=pl.ANY),
            pl.BlockSpec(memory_space=pl.ANY),
        ),
        scratch_shapes=[
            pltpu.VMEM((t, cw), jnp.bfloat16),
            pltpu.SemaphoreType.DMA,
            pltpu.SemaphoreType.DMA((_N_CHUNK,)),
            pltpu.SemaphoreType.DMA((_N_CHUNK,)),
            pltpu.SemaphoreType.DMA((_N_CHUNK,)),
            pltpu.SemaphoreType.DMA((_N_CHUNK,)),
        ],
        compiler_params=pltpu.CompilerParams(
            collective_id=0,
            vmem_limit_bytes=60 * 1024 * 1024,
            dimension_semantics=("arbitrary",),
        ),
    )(x, W)


def _softmax_two_halves(local, other):
    J, t, cw = local.shape
    v = J * cw
    tm = 64

    def body(l_ref, o_ref, out_ref):
        my_x = lax.axis_index("x")

        def load(ref, j):
            return ref[j].astype(jnp.float32)

        m = jnp.max(load(l_ref, 0), axis=-1, keepdims=True)
        for j in range(1, J):
            m = jnp.maximum(m, jnp.max(load(l_ref, j), axis=-1, keepdims=True))
        for j in range(J):
            m = jnp.maximum(m, jnp.max(load(o_ref, j), axis=-1, keepdims=True))

        s = jnp.zeros((tm, 1), jnp.float32)
        off_l = my_x * v
        off_o = (1 - my_x) * v
        for j in range(J):
            e = jnp.exp(load(l_ref, j) - m)
            s = s + jnp.sum(e, axis=-1, keepdims=True)
            out_ref[:, pl.ds(off_l + j * cw, cw)] = e
        for j in range(J):
            e = jnp.exp(load(o_ref, j) - m)
            s = s + jnp.sum(e, axis=-1, keepdims=True)
            out_ref[:, pl.ds(off_o + j * cw, cw)] = e
        out_ref[...] = out_ref[...] * (1.0 / s)

    return pl.pallas_call(
        body,
        grid=(t // tm,),
        in_specs=[
            pl.BlockSpec((J, tm, cw), lambda i: (0, i, 0)),
            pl.BlockSpec((J, tm, cw), lambda i: (0, i, 0)),
        ],
        out_specs=pl.BlockSpec((tm, 2 * v), lambda i: (i, 0)),
        out_shape=jax.ShapeDtypeStruct((t, 2 * v), jnp.float32),
    )(local, other)


_USE_FUSED = True


def kernel(x, W):
    if _USE_FUSED:
        logits, other = _gemm_exchange(x.astype(jnp.bfloat16), W)
        return _softmax_two_halves(logits, other)
    logits = jnp.dot(x, W, preferred_element_type=jnp.float32)
    other = _exchange(logits)
    t, v = logits.shape
    cw = v // _N_CHUNK
    lc = jnp.swapaxes(logits.reshape(t, _N_CHUNK, cw), 0, 1)
    oc = jnp.swapaxes(other.reshape(t, _N_CHUNK, cw), 0, 1)
    return _softmax_two_halves(lc, oc)


# device time: 305014 ns/iter; 1.0440x vs baseline; 1.0440x over previous
import jax
import jax.numpy as jnp
from jax import lax
from jax.experimental import pallas as pl
from jax.experimental.pallas import tpu as pltpu


_N_CHUNK = 16


def _exchange(logits):
    t, v = logits.shape
    half = t // 2
    rows = half // _N_CHUNK

    def body(l_ref, o_ref, x_send, x_recv, y_send, y_recv):
        my_x = lax.axis_index("x")
        my_y = lax.axis_index("y")
        xn = (1 - my_x, my_y)
        yn = (my_x, 1 - my_y)

        barrier = pltpu.get_barrier_semaphore()
        for nbr in (xn, yn):
            pl.semaphore_signal(
                barrier, inc=1, device_id=nbr, device_id_type=pl.DeviceIdType.MESH
            )
        pl.semaphore_wait(barrier, 2)

        base = my_y * half

        x_rdmas = []
        for c in range(_N_CHUNK):
            r = pl.ds(base + c * rows, rows)
            rd = pltpu.make_async_remote_copy(
                src_ref=l_ref.at[r],
                dst_ref=o_ref.at[r],
                send_sem=x_send.at[c],
                recv_sem=x_recv.at[c],
                device_id=xn,
                device_id_type=pl.DeviceIdType.MESH,
            )
            rd.start()
            x_rdmas.append(rd)

        y_rdmas = []
        for c in range(_N_CHUNK):
            x_rdmas[c].wait_recv()
            r = pl.ds(base + c * rows, rows)
            rd = pltpu.make_async_remote_copy(
                src_ref=o_ref.at[r],
                dst_ref=o_ref.at[r],
                send_sem=y_send.at[c],
                recv_sem=y_recv.at[c],
                device_id=yn,
                device_id_type=pl.DeviceIdType.MESH,
            )
            rd.start()
            y_rdmas.append(rd)

        for c in range(_N_CHUNK):
            y_rdmas[c].wait_recv()
            x_rdmas[c].wait_send()
            y_rdmas[c].wait_send()

    return pl.pallas_call(
        body,
        out_shape=jax.ShapeDtypeStruct((t, v), jnp.float32),
        in_specs=[pl.BlockSpec(memory_space=pl.ANY)],
        out_specs=pl.BlockSpec(memory_space=pl.ANY),
        scratch_shapes=[
            pltpu.SemaphoreType.DMA((_N_CHUNK,)),
            pltpu.SemaphoreType.DMA((_N_CHUNK,)),
            pltpu.SemaphoreType.DMA((_N_CHUNK,)),
            pltpu.SemaphoreType.DMA((_N_CHUNK,)),
        ],
        compiler_params=pltpu.CompilerParams(collective_id=0),
    )(logits)


def _gemm_exchange(x, W):
    t, d = x.shape
    v = W.shape[1]
    half = t // 2
    cw = v // _N_CHUNK
    J = _N_CHUNK

    def body(x_ref, w_ref, l_out, o_out, acc, cp_sem, x_send, x_recv, y_send, y_recv):
        j = pl.program_id(0)
        my_x = lax.axis_index("x")
        my_y = lax.axis_index("y")
        xn = (1 - my_x, my_y)
        yn = (my_x, 1 - my_y)
        base = my_y * half
        rs = pl.ds(base, half)

        @pl.when(j == 0)
        def _():
            barrier = pltpu.get_barrier_semaphore()
            for nbr in (xn, yn):
                pl.semaphore_signal(
                    barrier, inc=1, device_id=nbr,
                    device_id_type=pl.DeviceIdType.MESH,
                )
            pl.semaphore_wait(barrier, 2)

        def x_rdma(k):
            return pltpu.make_async_remote_copy(
                src_ref=l_out.at[k, rs],
                dst_ref=o_out.at[k, rs],
                send_sem=x_send.at[k],
                recv_sem=x_recv.at[k],
                device_id=xn,
                device_id_type=pl.DeviceIdType.MESH,
            )

        def y_rdma(k):
            return pltpu.make_async_remote_copy(
                src_ref=o_out.at[k, rs],
                dst_ref=o_out.at[k, rs],
                send_sem=y_send.at[k],
                recv_sem=y_recv.at[k],
                device_id=yn,
                device_id_type=pl.DeviceIdType.MESH,
            )

        @pl.when(j >= 1)
        def _():
            pltpu.make_async_copy(acc, l_out.at[j - 1], cp_sem).wait()
            x_rdma(j - 1).start()

        @pl.when(j >= 3)
        def _():
            k = j - 3
            x_rdma(k).wait_recv()
            y_rdma(k).start()

        acc[...] = jnp.dot(
            x_ref[...], w_ref[...], preferred_element_type=jnp.float32
        ).astype(jnp.bfloat16)
        cp = pltpu.make_async_copy(acc, l_out.at[j], cp_sem)
        cp.start()

        @pl.when(j == J - 1)
        def _():
            pltpu.make_async_copy(acc, l_out.at[J - 1], cp_sem).wait()
            x_rdma(J - 1).start()
            for k in (J - 3, J - 2, J - 1):
                x_rdma(k).wait_recv()
                y_rdma(k).start()
            for k in range(J):
                y_rdma(k).wait_recv()
                x_rdma(k).wait_send()
                y_rdma(k).wait_send()

    return pl.pallas_call(
        body,
        grid=(J,),
        out_shape=(
            jax.ShapeDtypeStruct((J, t, cw), jnp.bfloat16),
            jax.ShapeDtypeStruct((J, t, cw), jnp.bfloat16),
        ),
        in_specs=[
            pl.BlockSpec((t, d), lambda j: (0, 0)),
            pl.BlockSpec((d, cw), lambda j: (0, j)),
        ],
        out_specs=(
            pl.BlockSpec(memory_space=pl.ANY),
            pl.BlockSpec(memory_space=pl.ANY),
        ),
        scratch_shapes=[
            pltpu.VMEM((t, cw), jnp.bfloat16),
            pltpu.SemaphoreType.DMA,
            pltpu.SemaphoreType.DMA((_N_CHUNK,)),
            pltpu.SemaphoreType.DMA((_N_CHUNK,)),
            pltpu.SemaphoreType.DMA((_N_CHUNK,)),
            pltpu.SemaphoreType.DMA((_N_CHUNK,)),
        ],
        compiler_params=pltpu.CompilerParams(
            collective_id=0,
            vmem_limit_bytes=60 * 1024 * 1024,
            dimension_semantics=("arbitrary",),
        ),
    )(x, W)


def _softmax_two_halves(local, other):
    J, t, cw = local.shape
    v = J * cw
    tm = 64

    def body(l_ref, o_ref, out_ref):
        my_x = lax.axis_index("x")

        def load(ref, j):
            return ref[j].astype(jnp.float32)

        m = jnp.max(load(l_ref, 0), axis=-1, keepdims=True)
        for j in range(1, J):
            m = jnp.maximum(m, jnp.max(load(l_ref, j), axis=-1, keepdims=True))
        for j in range(J):
            m = jnp.maximum(m, jnp.max(load(o_ref, j), axis=-1, keepdims=True))

        s = jnp.zeros((tm, 1), jnp.float32)
        off_l = my_x * v
        off_o = (1 - my_x) * v
        for j in range(J):
            e = jnp.exp(load(l_ref, j) - m)
            s = s + jnp.sum(e, axis=-1, keepdims=True)
            out_ref[:, pl.ds(off_l + j * cw, cw)] = e
        for j in range(J):
            e = jnp.exp(load(o_ref, j) - m)
            s = s + jnp.sum(e, axis=-1, keepdims=True)
            out_ref[:, pl.ds(off_o + j * cw, cw)] = e
        out_ref[...] = out_ref[...] * (1.0 / s)

    return pl.pallas_call(
        body,
        grid=(t // tm,),
        in_specs=[
            pl.BlockSpec((J, tm, cw), lambda i: (0, i, 0)),
            pl.BlockSpec((J, tm, cw), lambda i: (0, i, 0)),
        ],
        out_specs=pl.BlockSpec((tm, 2 * v), lambda i: (i, 0)),
        out_shape=jax.ShapeDtypeStruct((t, 2 * v), jnp.float32),
    )(local, other)


_USE_FUSED = True


def kernel(x, W):
    if _USE_FUSED:
        logits, other = _gemm_exchange(x, W)
        return _softmax_two_halves(logits, other)
    logits = jnp.dot(x, W, preferred_element_type=jnp.float32)
    other = _exchange(logits)
    t, v = logits.shape
    cw = v // _N_CHUNK
    lc = jnp.swapaxes(logits.reshape(t, _N_CHUNK, cw), 0, 1)
    oc = jnp.swapaxes(other.reshape(t, _N_CHUNK, cw), 0, 1)
    return _softmax_two_halves(lc, oc)
